# baseline (speedup 1.0000x reference)
import functools

import jax
import jax.numpy as jnp
from jax import lax
from jax.experimental import pallas as pl
from jax.experimental.pallas import tpu as pltpu
from jax.experimental.pallas import tpu_sc as plsc

V = 1_000_000
DIM = 64
N_CTX = 200
N_NEG = 12800
N_REAL = N_CTX + N_NEG
NC, NS = 2, 16
NW = NC * NS
IPW = 512
BTOT = NW * IPW
NCH = IPW // 128
NB = 65536
GRID = (V + NB - 1) // NB


def _mv_body(scal_ref, in_t_ref, out_t_ref, y_ref):
    tmod = scal_ref[1]
    onehot = (lax.broadcasted_iota(jnp.int32, (128, 1), 0) == tmod)
    t_col = lax.dot_general(in_t_ref[...], onehot.astype(jnp.float32),
                            (((1,), (0,)), ((), ())),
                            preferred_element_type=jnp.float32)
    s = lax.dot_general(t_col, out_t_ref[...], (((0,), (0,)), ((), ())),
                        preferred_element_type=jnp.float32)
    y_ref[...] = s.reshape((NB,))


def _matvec(in_t, out_t, scal):
    grid_spec = pltpu.PrefetchScalarGridSpec(
        num_scalar_prefetch=1,
        grid=(GRID,),
        in_specs=[
            pl.BlockSpec((DIM, 128), lambda i, s: (0, s[0])),
            pl.BlockSpec((DIM, NB), lambda i, s: (0, i)),
        ],
        out_specs=pl.BlockSpec((NB,), lambda i, s: (i,)),
    )
    return pl.pallas_call(
        _mv_body,
        grid_spec=grid_spec,
        out_shape=jax.ShapeDtypeStruct((GRID * NB,), jnp.float32),
        compiler_params=pltpu.CompilerParams(
            vmem_limit_bytes=50 * 1024 * 1024),
    )(scal, in_t, out_t)


def _sc_body(y_hbm, idx_hbm, sign_hbm, out_hbm, idx_v, sign_v, g_v, acc_v,
             sem):
    wid = lax.axis_index("s") * NC + lax.axis_index("c")
    d_idx = pltpu.async_copy(idx_hbm.at[wid], idx_v, sem)
    d_sgn = pltpu.async_copy(sign_hbm.at[wid], sign_v, sem)
    d_idx.wait()
    descs = [
        pltpu.async_copy(y_hbm.at[idx_v.at[k]],
                         g_v.at[pl.ds(k * 128, 128)], sem)
        for k in range(NCH)
    ]
    d_sgn.wait()
    for dsc in descs:
        dsc.wait()

    tot = jnp.zeros((16,), jnp.float32)
    for c in range(IPW // 16):
        sgn = sign_v[pl.ds(c * 16, 16)]
        z = g_v[pl.ds(c * 16, 16)] * sgn
        u = jnp.exp(-jnp.abs(z))
        y = u / (2.0 + u)
        y2 = y * y
        l1p = y * (2.0 + y2 * (2.0 / 3.0 + y2 * (2.0 / 5.0 + y2 * (
            2.0 / 7.0 + y2 * (2.0 / 9.0 + y2 * (2.0 / 11.0))))))
        contrib = jnp.minimum(z, 0.0) - l1p
        contrib = jnp.where(sgn == 0.0, 0.0, contrib)
        tot = tot + contrib
    acc_v[...] = tot
    pltpu.sync_copy(acc_v, out_hbm.at[pl.ds(wid * 16, 16)])


_sc_reduce = functools.partial(
    pl.kernel,
    mesh=plsc.VectorSubcoreMesh(core_axis_name="c", subcore_axis_name="s"),
    compiler_params=pltpu.CompilerParams(use_tc_tiling_on_sc=False),
    out_type=jax.ShapeDtypeStruct((NW * 16,), jnp.float32),
    scratch_types=[
        pltpu.VMEM((NCH, 128), jnp.int32),
        pltpu.VMEM((IPW,), jnp.float32),
        pltpu.VMEM((IPW,), jnp.float32),
        pltpu.VMEM((16,), jnp.float32),
        pltpu.SemaphoreType.DMA,
    ],
)(_sc_body)


def kernel(input_embeddings, output_embeddings, target, context,
           negative_samples):
    tgt = jnp.asarray(target, jnp.int32)
    scal = jnp.stack([tgt // 128, tgt % 128])
    y = _matvec(input_embeddings.T, output_embeddings.T, scal)

    idx = jnp.concatenate([
        context.astype(jnp.int32),
        negative_samples.reshape(-1).astype(jnp.int32),
        jnp.zeros((BTOT - N_REAL,), jnp.int32),
    ]).reshape(NW, NCH, 128)
    sign = jnp.concatenate([
        jnp.ones((N_CTX,), jnp.float32),
        jnp.full((N_NEG,), -1.0, jnp.float32),
        jnp.zeros((BTOT - N_REAL,), jnp.float32),
    ]).reshape(NW, IPW)

    partials = _sc_reduce(y, idx, sign)
    return -jnp.sum(partials)


# speedup vs baseline: 1.0147x; 1.0147x over previous
import functools

import jax
import jax.numpy as jnp
from jax import lax
from jax.experimental import pallas as pl
from jax.experimental.pallas import tpu as pltpu
from jax.experimental.pallas import tpu_sc as plsc

V = 1_000_000
DIM = 64
N_CTX = 200
N_NEG = 12800
N_REAL = N_CTX + N_NEG
NC, NS = 2, 16
NW = NC * NS
IPW = 512
BTOT = NW * IPW
NCH = IPW // 128
NB = 32768
GRID = (V + NB - 1) // NB


def _mv_body(scal_ref, in_t_ref, out_t_ref, y_ref):
    tmod = scal_ref[1]
    onehot = (lax.broadcasted_iota(jnp.int32, (128, 1), 0) == tmod)
    t_col = lax.dot_general(in_t_ref[...], onehot.astype(jnp.float32),
                            (((1,), (0,)), ((), ())),
                            preferred_element_type=jnp.float32)
    s = lax.dot_general(t_col, out_t_ref[...], (((0,), (0,)), ((), ())),
                        preferred_element_type=jnp.float32)
    y_ref[...] = s.reshape((NB,))


def _matvec(in_t, out_t, scal):
    grid_spec = pltpu.PrefetchScalarGridSpec(
        num_scalar_prefetch=1,
        grid=(GRID,),
        in_specs=[
            pl.BlockSpec((DIM, 128), lambda i, s: (0, s[0])),
            pl.BlockSpec((DIM, NB), lambda i, s: (0, i)),
        ],
        out_specs=pl.BlockSpec((NB,), lambda i, s: (i,)),
    )
    return pl.pallas_call(
        _mv_body,
        grid_spec=grid_spec,
        out_shape=jax.ShapeDtypeStruct((GRID * NB,), jnp.float32),
        compiler_params=pltpu.CompilerParams(
            vmem_limit_bytes=50 * 1024 * 1024),
    )(scal, in_t, out_t)


def _sc_body(y_hbm, idx_hbm, sign_hbm, out_hbm, idx_v, sign_v, g_v, acc_v,
             sem):
    wid = lax.axis_index("s") * NC + lax.axis_index("c")
    d_idx = pltpu.async_copy(idx_hbm.at[wid], idx_v, sem)
    d_sgn = pltpu.async_copy(sign_hbm.at[wid], sign_v, sem)
    d_idx.wait()
    descs = [
        pltpu.async_copy(y_hbm.at[idx_v.at[k]],
                         g_v.at[pl.ds(k * 128, 128)], sem)
        for k in range(NCH)
    ]
    d_sgn.wait()
    for dsc in descs:
        dsc.wait()

    tot = jnp.zeros((16,), jnp.float32)
    for c in range(IPW // 16):
        sgn = sign_v[pl.ds(c * 16, 16)]
        z = g_v[pl.ds(c * 16, 16)] * sgn
        u = jnp.exp(-jnp.abs(z))
        y = u / (2.0 + u)
        y2 = y * y
        l1p = y * (2.0 + y2 * (2.0 / 3.0 + y2 * (2.0 / 5.0 + y2 * (
            2.0 / 7.0 + y2 * (2.0 / 9.0 + y2 * (2.0 / 11.0))))))
        contrib = jnp.minimum(z, 0.0) - l1p
        contrib = jnp.where(sgn == 0.0, 0.0, contrib)
        tot = tot + contrib
    acc_v[...] = tot
    pltpu.sync_copy(acc_v, out_hbm.at[pl.ds(wid * 16, 16)])


_sc_reduce = functools.partial(
    pl.kernel,
    mesh=plsc.VectorSubcoreMesh(core_axis_name="c", subcore_axis_name="s"),
    compiler_params=pltpu.CompilerParams(use_tc_tiling_on_sc=False),
    out_type=jax.ShapeDtypeStruct((NW * 16,), jnp.float32),
    scratch_types=[
        pltpu.VMEM((NCH, 128), jnp.int32),
        pltpu.VMEM((IPW,), jnp.float32),
        pltpu.VMEM((IPW,), jnp.float32),
        pltpu.VMEM((16,), jnp.float32),
        pltpu.SemaphoreType.DMA,
    ],
)(_sc_body)


def kernel(input_embeddings, output_embeddings, target, context,
           negative_samples):
    tgt = jnp.asarray(target, jnp.int32)
    scal = jnp.stack([tgt // 128, tgt % 128])
    y = _matvec(input_embeddings.T, output_embeddings.T, scal)

    idx = jnp.concatenate([
        context.astype(jnp.int32),
        negative_samples.reshape(-1).astype(jnp.int32),
        jnp.zeros((BTOT - N_REAL,), jnp.int32),
    ]).reshape(NW, NCH, 128)
    sign = jnp.concatenate([
        jnp.ones((N_CTX,), jnp.float32),
        jnp.full((N_NEG,), -1.0, jnp.float32),
        jnp.zeros((BTOT - N_REAL,), jnp.float32),
    ]).reshape(NW, IPW)

    partials = _sc_reduce(y, idx, sign)
    return -jnp.sum(partials)
